# baseline (speedup 1.0000x reference)
import jax
import jax.numpy as jnp
from jax import lax
from jax.experimental import pallas as pl
from jax.experimental.pallas import tpu as pltpu
from jax.experimental.pallas import tpu_sc as plsc

_BATCH = 16384
_EMBD = 32
_NW = 32
_BPW = _BATCH // _NW
_CHUNK = 128
_NCH = _BPW // _CHUNK


def _pmf_body(uid_hbm, iid_hbm, ut_hbm, it_hbm, bu_hbm, bi_hbm, out_hbm,
              uid_v, iid_v, u_t, i_t, bu_v, bi_v, out_v, sem):
    wid = lax.axis_index("s") * 2 + lax.axis_index("c")

    pltpu.sync_copy(uid_hbm.at[wid], uid_v)
    pltpu.sync_copy(iid_hbm.at[wid], iid_v)

    copies = []
    for j in range(_NCH):
        sl = pl.ds(j * _CHUNK, _CHUNK)
        copies.append(pltpu.async_copy(bu_hbm.at[uid_v.at[j]], bu_v.at[sl], sem))
        copies.append(pltpu.async_copy(bi_hbm.at[iid_v.at[j]], bi_v.at[sl], sem))
    for d in range(_EMBD):
        for j in range(_NCH):
            sl = pl.ds(j * _CHUNK, _CHUNK)
            copies.append(
                pltpu.async_copy(ut_hbm.at[d].at[uid_v.at[j]], u_t.at[d, sl], sem))
            copies.append(
                pltpu.async_copy(it_hbm.at[d].at[iid_v.at[j]], i_t.at[d, sl], sem))
    for c in copies:
        c.wait()

    def chunk_body(c, carry):
        sl = pl.ds(c * 16, 16)
        acc = bu_v[sl] + bi_v[sl]
        for d in range(_EMBD):
            acc = acc + u_t[d, sl] * i_t[d, sl]
        out_v[sl] = acc
        return carry

    lax.fori_loop(0, _BPW // 16, chunk_body, 0)

    pltpu.sync_copy(out_v, out_hbm.at[wid])


@jax.jit
def _pmf(uid, iid, ut_t, it_t, bu_f, bi_f):
    mesh = plsc.VectorSubcoreMesh(core_axis_name="c", subcore_axis_name="s")
    kfn = pl.kernel(
        _pmf_body,
        out_type=jax.ShapeDtypeStruct((_NW, _BPW), jnp.float32),
        mesh=mesh,
        scratch_types=[
            pltpu.VMEM((_NCH, _CHUNK), jnp.int32),
            pltpu.VMEM((_NCH, _CHUNK), jnp.int32),
            pltpu.VMEM((_EMBD, _BPW), jnp.float32),
            pltpu.VMEM((_EMBD, _BPW), jnp.float32),
            pltpu.VMEM((_BPW,), jnp.float32),
            pltpu.VMEM((_BPW,), jnp.float32),
            pltpu.VMEM((_BPW,), jnp.float32),
            pltpu.SemaphoreType.DMA,
        ],
        compiler_params=pltpu.CompilerParams(
            use_tc_tiling_on_sc=False,
        ),
        name="pmf_sc",
    )
    return kfn(uid, iid, ut_t, it_t, bu_f, bi_f)


def kernel(user_review, item_review, uid, iid, user_table, item_table,
           b_users, b_items, b_0):
    del user_review, item_review
    uid = uid.astype(jnp.int32).reshape(_NW, _NCH, _CHUNK)
    iid = iid.astype(jnp.int32).reshape(_NW, _NCH, _CHUNK)
    out = _pmf(uid, iid, user_table.T, item_table.T,
               b_users.reshape(-1), b_items.reshape(-1))
    return out.reshape(_BATCH) + b_0[0]


# speedup vs baseline: 14.9777x; 14.9777x over previous
import jax
import jax.numpy as jnp
from jax import lax
from jax.experimental import pallas as pl
from jax.experimental.pallas import tpu as pltpu
from jax.experimental.pallas import tpu_sc as plsc

_BATCH = 16384
_EMBD = 32
_NW = 32
_BPW = _BATCH // _NW
_CHUNK = 128
_NCH = _BPW // _CHUNK
_GRP = 4
_NG = _BPW // _GRP

def _pmf_body(uid_hbm, iid_hbm, ut_hbm, it_hbm, bu_hbm, bi_hbm, out_hbm,
              uid_v, iid_v, slot_u, slot_i, slot_bu, slot_bi,
              u_t, i_t, bu_v, bi_v, out_v, sems):
    wid = lax.axis_index("s") * 2 + lax.axis_index("c")

    pltpu.sync_copy(uid_hbm.at[wid], uid_v)
    pltpu.sync_copy(iid_hbm.at[wid], iid_v)

    d_lo = lax.iota(jnp.int32, 16)
    d_hi = d_lo + 16
    lane0 = d_lo == 0

    def idx_vecs(g):
        b = g * _GRP
        j = b // _CHUNK
        sl16 = pl.ds(((b % _CHUNK) // 16) * 16, 16)
        return uid_v[j, sl16], iid_v[j, sl16]

    def issue_group(g, lo, bank):
        u16, i16 = idx_vecs(g)
        sem = sems.at[bank]
        for l in range(_GRP):
            u = u16[lo + l]
            i = i16[lo + l]
            qu = pl.multiple_of((u >> 7) * 128, 128)
            qi = pl.multiple_of((i >> 7) * 128, 128)
            s = bank * _GRP + l
            pltpu.async_copy(ut_hbm.at[:, pl.ds(qu, 128)], slot_u.at[s], sem)
            pltpu.async_copy(it_hbm.at[:, pl.ds(qi, 128)], slot_i.at[s], sem)
            pltpu.async_copy(bu_hbm.at[pl.ds(qu, 128)], slot_bu.at[s], sem)
            pltpu.async_copy(bi_hbm.at[pl.ds(qi, 128)], slot_bi.at[s], sem)

    def wait_group(bank):
        sem = sems.at[bank]
        for l in range(_GRP):
            s = bank * _GRP + l
            pltpu.make_async_copy(ut_hbm.at[:, pl.ds(0, 128)], slot_u.at[s], sem).wait()
            pltpu.make_async_copy(it_hbm.at[:, pl.ds(0, 128)], slot_i.at[s], sem).wait()
            pltpu.make_async_copy(bu_hbm.at[pl.ds(0, 128)], slot_bu.at[s], sem).wait()
            pltpu.make_async_copy(bi_hbm.at[pl.ds(0, 128)], slot_bi.at[s], sem).wait()

    def extract_group(g, lo, bank):
        u16, i16 = idx_vecs(g)
        base = g * _GRP
        for l in range(_GRP):
            s = bank * _GRP + l
            b = base + l
            ru = jnp.full((16,), u16[lo + l] & 127, jnp.int32)
            ri = jnp.full((16,), i16[lo + l] & 127, jnp.int32)
            bcol = jnp.full((16,), b, jnp.int32)
            ulo = plsc.load_gather(slot_u.at[s], [d_lo, ru])
            uhi = plsc.load_gather(slot_u.at[s], [d_hi, ru])
            ilo = plsc.load_gather(slot_i.at[s], [d_lo, ri])
            ihi = plsc.load_gather(slot_i.at[s], [d_hi, ri])
            plsc.store_scatter(u_t, [d_lo, bcol], ulo)
            plsc.store_scatter(u_t, [d_hi, bcol], uhi)
            plsc.store_scatter(i_t, [d_lo, bcol], ilo)
            plsc.store_scatter(i_t, [d_hi, bcol], ihi)
            bu16 = plsc.load_gather(slot_bu.at[s], [ru])
            bi16 = plsc.load_gather(slot_bi.at[s], [ri])
            plsc.store_scatter(bu_v, [bcol], bu16, mask=lane0)
            plsc.store_scatter(bi_v, [bcol], bi16, mask=lane0)

    def pipe_body(t, carry):
        g0 = 4 * t
        issue_group(g0, 0, 0)

        @pl.when(t >= 1)
        def _():
            wait_group(1)
            extract_group(g0 - 1, 12, 1)

        issue_group(g0 + 1, 4, 1)
        wait_group(0)
        extract_group(g0, 0, 0)
        issue_group(g0 + 2, 8, 0)
        wait_group(1)
        extract_group(g0 + 1, 4, 1)
        issue_group(g0 + 3, 12, 1)
        wait_group(0)
        extract_group(g0 + 2, 8, 0)
        return carry

    lax.fori_loop(0, _NG // 4, pipe_body, 0)
    wait_group(1)
    extract_group(_NG - 1, 12, 1)

    def chunk_body(c, carry):
        sl = pl.ds(c * 16, 16)
        acc = bu_v[sl] + bi_v[sl]
        for d in range(_EMBD):
            acc = acc + u_t[d, sl] * i_t[d, sl]
        out_v[sl] = acc
        return carry

    lax.fori_loop(0, _BPW // 16, chunk_body, 0)

    pltpu.sync_copy(out_v, out_hbm.at[wid])


@jax.jit
def _pmf(uid, iid, ut_t, it_t, bu_f, bi_f):
    mesh = plsc.VectorSubcoreMesh(core_axis_name="c", subcore_axis_name="s")
    kfn = pl.kernel(
        _pmf_body,
        out_type=jax.ShapeDtypeStruct((_NW, _BPW), jnp.float32),
        mesh=mesh,
        scratch_types=[
            pltpu.VMEM((_NCH, _CHUNK), jnp.int32),
            pltpu.VMEM((_NCH, _CHUNK), jnp.int32),
            pltpu.VMEM((2 * _GRP, _EMBD, 128), jnp.float32),
            pltpu.VMEM((2 * _GRP, _EMBD, 128), jnp.float32),
            pltpu.VMEM((2 * _GRP, 128), jnp.float32),
            pltpu.VMEM((2 * _GRP, 128), jnp.float32),
            pltpu.VMEM((_EMBD, _BPW), jnp.float32),
            pltpu.VMEM((_EMBD, _BPW), jnp.float32),
            pltpu.VMEM((_BPW,), jnp.float32),
            pltpu.VMEM((_BPW,), jnp.float32),
            pltpu.VMEM((_BPW,), jnp.float32),
            pltpu.SemaphoreType.DMA((2,)),
        ],
        compiler_params=pltpu.CompilerParams(
            use_tc_tiling_on_sc=True,
            needs_layout_passes=False,
            disable_bounds_checks=True,
        ),
        name="pmf_sc",
    )
    return kfn(uid, iid, ut_t, it_t, bu_f, bi_f)


def kernel(user_review, item_review, uid, iid, user_table, item_table,
           b_users, b_items, b_0):
    del user_review, item_review
    uid = uid.astype(jnp.int32).reshape(_NW, _NCH, _CHUNK)
    iid = iid.astype(jnp.int32).reshape(_NW, _NCH, _CHUNK)
    out = _pmf(uid, iid, user_table.T, item_table.T,
               b_users.reshape(-1), b_items.reshape(-1))
    return out.reshape(_BATCH) + b_0[0]


# speedup vs baseline: 15.0551x; 1.0052x over previous
import jax
import jax.numpy as jnp
from jax import lax
from jax.experimental import pallas as pl
from jax.experimental.pallas import tpu as pltpu
from jax.experimental.pallas import tpu_sc as plsc

_BATCH = 16384
_EMBD = 32
_NW = 32
_BPW = _BATCH // _NW
_CHUNK = 128
_NCH = _BPW // _CHUNK
_GRP = 4
_NG = _BPW // _GRP

def _pmf_body(uid_hbm, iid_hbm, ut_hbm, it_hbm, bu_hbm, bi_hbm, out_hbm,
              uid_v, iid_v, slot_u, slot_i, slot_bu, slot_bi,
              u_t, i_t, bu_v, bi_v, out_v, sems):
    wid = lax.axis_index("s") * 2 + lax.axis_index("c")

    pltpu.sync_copy(uid_hbm.at[wid], uid_v)
    pltpu.sync_copy(iid_hbm.at[wid], iid_v)

    d_lo = lax.iota(jnp.int32, 16)
    d_hi = d_lo + 16
    lane0 = d_lo == 0

    def idx_vecs(g):
        b = g * _GRP
        j = b // _CHUNK
        sl16 = pl.ds(((b % _CHUNK) // 16) * 16, 16)
        return uid_v[j, sl16], iid_v[j, sl16]

    def issue_group(g, lo, bank):
        u16, i16 = idx_vecs(g)
        sem = sems.at[bank]
        for l in range(_GRP):
            u = u16[lo + l]
            i = i16[lo + l]
            qu = pl.multiple_of((u >> 7) * 128, 128)
            qi = pl.multiple_of((i >> 7) * 128, 128)
            s = bank * _GRP + l
            pltpu.async_copy(ut_hbm.at[:, pl.ds(qu, 128)], slot_u.at[s], sem)
            pltpu.async_copy(it_hbm.at[:, pl.ds(qi, 128)], slot_i.at[s], sem)
            pltpu.async_copy(bu_hbm.at[pl.ds(qu, 128)], slot_bu.at[s], sem)
            pltpu.async_copy(bi_hbm.at[pl.ds(qi, 128)], slot_bi.at[s], sem)

    def wait_group(bank):
        sem = sems.at[bank]
        for l in range(_GRP):
            s = bank * _GRP + l
            pltpu.make_async_copy(ut_hbm.at[:, pl.ds(0, 128)], slot_u.at[s], sem).wait()
            pltpu.make_async_copy(it_hbm.at[:, pl.ds(0, 128)], slot_i.at[s], sem).wait()
            pltpu.make_async_copy(bu_hbm.at[pl.ds(0, 128)], slot_bu.at[s], sem).wait()
            pltpu.make_async_copy(bi_hbm.at[pl.ds(0, 128)], slot_bi.at[s], sem).wait()

    def extract_group(g, lo, bank):
        u16, i16 = idx_vecs(g)
        base = g * _GRP
        for l in range(_GRP):
            s = bank * _GRP + l
            b = base + l
            ru = jnp.full((16,), u16[lo + l] & 127, jnp.int32)
            ri = jnp.full((16,), i16[lo + l] & 127, jnp.int32)
            bcol = jnp.full((16,), b, jnp.int32)
            ulo = plsc.load_gather(slot_u.at[s], [d_lo, ru])
            uhi = plsc.load_gather(slot_u.at[s], [d_hi, ru])
            ilo = plsc.load_gather(slot_i.at[s], [d_lo, ri])
            ihi = plsc.load_gather(slot_i.at[s], [d_hi, ri])
            plsc.store_scatter(u_t, [d_lo, bcol], ulo)
            plsc.store_scatter(u_t, [d_hi, bcol], uhi)
            plsc.store_scatter(i_t, [d_lo, bcol], ilo)
            plsc.store_scatter(i_t, [d_hi, bcol], ihi)
            bu16 = plsc.load_gather(slot_bu.at[s], [ru])
            bi16 = plsc.load_gather(slot_bi.at[s], [ri])
            plsc.store_scatter(bu_v, [bcol], bu16, mask=lane0)
            plsc.store_scatter(bi_v, [bcol], bi16, mask=lane0)

    def pipe_body(t, carry):
        g0 = 4 * t
        issue_group(g0, 0, 0)

        @pl.when(t >= 1)
        def _():
            wait_group(1)
            extract_group(g0 - 1, 12, 1)

        issue_group(g0 + 1, 4, 1)
        wait_group(0)
        extract_group(g0, 0, 0)
        issue_group(g0 + 2, 8, 0)
        wait_group(1)
        extract_group(g0 + 1, 4, 1)
        issue_group(g0 + 3, 12, 1)
        wait_group(0)
        extract_group(g0 + 2, 8, 0)
        return carry

    lax.fori_loop(0, _NG // 4, pipe_body, 0)
    wait_group(1)
    extract_group(_NG - 1, 12, 1)

    def chunk_body(c, carry):
        sl = pl.ds(c * 16, 16)
        acc = bu_v[sl] + bi_v[sl]
        for d in range(_EMBD):
            acc = acc + u_t[d, sl] * i_t[d, sl]
        out_v[sl] = acc
        return carry

    lax.fori_loop(0, _BPW // 16, chunk_body, 0)

    pltpu.sync_copy(out_v, out_hbm.at[wid])


@jax.jit
def _pmf(uid, iid, ut_t, it_t, bu_f, bi_f):
    mesh = plsc.VectorSubcoreMesh(core_axis_name="c", subcore_axis_name="s")
    kfn = pl.kernel(
        _pmf_body,
        out_type=jax.ShapeDtypeStruct((_NW, _BPW), jnp.float32),
        mesh=mesh,
        scratch_types=[
            pltpu.VMEM((_NCH, _CHUNK), jnp.int32),
            pltpu.VMEM((_NCH, _CHUNK), jnp.int32),
            pltpu.VMEM((2 * _GRP, _EMBD, 128), jnp.float32),
            pltpu.VMEM((2 * _GRP, _EMBD, 128), jnp.float32),
            pltpu.VMEM((2 * _GRP, 128), jnp.float32),
            pltpu.VMEM((2 * _GRP, 128), jnp.float32),
            pltpu.VMEM((_EMBD, _BPW), jnp.float32),
            pltpu.VMEM((_EMBD, _BPW), jnp.float32),
            pltpu.VMEM((_BPW,), jnp.float32),
            pltpu.VMEM((_BPW,), jnp.float32),
            pltpu.VMEM((_BPW,), jnp.float32),
            pltpu.SemaphoreType.DMA((2,)),
        ],
        compiler_params=pltpu.CompilerParams(
            use_tc_tiling_on_sc=True,
            needs_layout_passes=False,
            disable_bounds_checks=True,
            skip_device_barrier=True,
        ),
        name="pmf_sc",
    )
    return kfn(uid, iid, ut_t, it_t, bu_f, bi_f)


def kernel(user_review, item_review, uid, iid, user_table, item_table,
           b_users, b_items, b_0):
    del user_review, item_review
    uid = uid.astype(jnp.int32).reshape(_NW, _NCH, _CHUNK)
    iid = iid.astype(jnp.int32).reshape(_NW, _NCH, _CHUNK)
    out = _pmf(uid, iid, user_table.T, item_table.T,
               b_users.reshape(-1), b_items.reshape(-1))
    return out.reshape(_BATCH) + b_0[0]
